# Initial kernel scaffold; baseline (speedup 1.0000x reference)
import jax
import jax.numpy as jnp
from jax.experimental import pallas as pl


def kernel(x_p, x_d, edge_attr_p, edge_attr_d, edge_index_p, edge_index_d, x_p_batch, x_d_batch, params):
    raise NotImplementedError("write your pallas kernel here")


# speedup vs baseline: 1.8321x; 1.8321x over previous
import functools

import jax
import jax.numpy as jnp
from jax import lax
from jax.experimental import pallas as pl
from jax.experimental.pallas import tpu as pltpu
from jax.experimental.pallas import tpu_sc as plsc

F32 = jnp.float32

NNET = 10000
PADN = 10048
NP = 2 * PADN
EDGES = 320000
EALL = 2 * EDGES
EPAD = EALL + 256
C = 128
NSUB = 16
HID = 32
DIN = 128
NG = 512
RB = 1256
GRID = NP // RB
NPB = GRID // 2
RMAX = 640


def _mid_body(x_ref, a_ref, w1_ref, b1_ref, w2_ref, b2_ref, o_ref, *, last):
    h = x_ref[...] + a_ref[...]
    t = jnp.maximum(jnp.dot(h, w1_ref[0], preferred_element_type=F32) + b1_ref[0], 0.0)
    u = jnp.dot(t, w2_ref[0], preferred_element_type=F32) + b2_ref[0]
    if not last:
        u = jnp.maximum(u, 0.0)
    o_ref[...] = u


def _mid(x, agg, w1, b1, w2, b2, last):
    din = x.shape[1]
    return pl.pallas_call(
        functools.partial(_mid_body, last=last),
        grid=(GRID,),
        in_specs=[
            pl.BlockSpec((RB, din), lambda i: (i, 0)),
            pl.BlockSpec((RB, din), lambda i: (i, 0)),
            pl.BlockSpec((1, din, HID), lambda i: (i // NPB, 0, 0)),
            pl.BlockSpec((1, 1, HID), lambda i: (i // NPB, 0, 0)),
            pl.BlockSpec((1, HID, HID), lambda i: (i // NPB, 0, 0)),
            pl.BlockSpec((1, 1, HID), lambda i: (i // NPB, 0, 0)),
        ],
        out_specs=pl.BlockSpec((RB, HID), lambda i: (i, 0)),
        out_shape=jax.ShapeDtypeStruct((NP, HID), F32),
    )(x, agg, w1, b1, w2, b2)


def _pool_body(x_ref, bat_ref, lw1_ref, lb1_ref, lw2_ref, lb2_ref,
               lw3_ref, lb3_ref, fw_ref, fb_ref, o_ref, s_ref, c_ref):
    i = pl.program_id(0)

    @pl.when(i == 0)
    def _init():
        s_ref[...] = jnp.zeros_like(s_ref)
        c_ref[...] = jnp.zeros_like(c_ref)

    x4 = x_ref[...]
    ids = bat_ref[0]
    gid = lax.broadcasted_iota(jnp.int32, (NG, RB), 0)
    oh = (gid == ids).astype(F32)
    s_ref[...] += jnp.dot(oh, x4, preferred_element_type=F32,
                          precision=lax.Precision.HIGHEST)
    c_ref[...] += jnp.sum(oh, axis=1, keepdims=True)

    @pl.when(i == GRID - 1)
    def _head():
        means = s_ref[...] / jnp.maximum(c_ref[...], 1.0)
        hp = means[:256]
        hd = means[256:]
        h = jnp.maximum(
            jnp.dot(hp, lw1_ref[:HID], preferred_element_type=F32)
            + jnp.dot(hd, lw1_ref[HID:], preferred_element_type=F32)
            + lb1_ref[...], 0.0)
        h = jnp.maximum(jnp.dot(h, lw2_ref[...], preferred_element_type=F32)
                        + lb2_ref[...], 0.0)
        h = jnp.maximum(jnp.dot(h, lw3_ref[...], preferred_element_type=F32)
                        + lb3_ref[...], 0.0)
        o_ref[...] = jnp.dot(h, fw_ref[...], preferred_element_type=F32) + fb_ref[...]


def _pool_head(x4, bat, lw1, lb1, lw2, lb2, lw3, lb3, fw, fb):
    full = lambda shape: pl.BlockSpec(shape, lambda i: tuple(0 for _ in shape))
    return pl.pallas_call(
        _pool_body,
        grid=(GRID,),
        in_specs=[
            pl.BlockSpec((RB, HID), lambda i: (i, 0)),
            pl.BlockSpec((1, 1, RB), lambda i: (i, 0, 0)),
            full((2 * HID, HID)),
            full((1, HID)),
            full((HID, HID)),
            full((1, HID)),
            full((HID, HID)),
            full((1, HID)),
            full((HID, 1)),
            full((1, 1)),
        ],
        out_specs=pl.BlockSpec((256, 1), lambda i: (0, 0)),
        out_shape=jax.ShapeDtypeStruct((256, 1), F32),
        scratch_shapes=[
            pltpu.VMEM((NG, HID), F32),
            pltpu.VMEM((NG, 1), F32),
        ],
    )(x4, bat, lw1, lb1, lw2, lb2, lw3, lb3, fw, fb)


def _mk_agg_body(F):
    NV = F // 16

    def body(y_hbm, src_hbm, dst_hbm, bnd_hbm, out_hbm, sidx, didx, bvm, rows, accv):
        c = lax.axis_index("c")
        s = lax.axis_index("s")
        wid = c * NSUB + s
        net = wid // NSUB
        tt = wid % NSUB
        dlo = net * PADN + tt * RMAX
        dhi = jnp.minimum(net * PADN + (tt + 1) * RMAX, net * PADN + PADN)
        nrows = dhi - dlo

        pltpu.sync_copy(bnd_hbm, bvm)
        z = jnp.zeros((16,), F32)

        def zloop(r, carry):
            accv[pl.ds(r * 16, 16)] = z
            return carry

        lax.fori_loop(0, RMAX * NV, zloop, 0)

        bv = bvm[pl.ds(wid, 16)]
        s_edge = bv[0]
        e_edge = bv[1]
        base0 = (s_edge // 8) * 8
        nchunks = (e_edge - base0 + C - 1) // C

        def chunk_body(k, carry):
            base = base0 + k * C
            pltpu.sync_copy(src_hbm.at[pl.ds(base, C)], sidx)
            pltpu.sync_copy(dst_hbm.at[pl.ds(base, C)], didx.at[pl.ds(0, C)])
            pltpu.sync_copy(y_hbm.at[sidx], rows)

            def edge_body(i, carry2):
                d_prev = carry2[0]
                acc = carry2[1:]
                d_i = didx[pl.ds(i, 16)][0]
                take = jnp.logical_and(d_i >= dlo, d_i < dhi)
                newseg = jnp.logical_and(take, d_i != d_prev)
                flush = jnp.logical_and(newseg, d_prev >= 0)

                @pl.when(flush)
                def _fl():
                    for j in range(NV):
                        accv[pl.ds((d_prev - dlo) * F + 16 * j, 16)] = acc[j]

                newacc = []
                for j in range(NV):
                    r = rows[i, pl.ds(16 * j, 16)]
                    a = jnp.where(newseg, r,
                                  jnp.where(take, acc[j] + r, acc[j]))
                    newacc.append(a)
                d_new = jnp.where(take, d_i, d_prev)
                return (d_new, *newacc)

            return lax.fori_loop(0, C, edge_body, carry)

        init = (jnp.int32(-1),) + tuple(jnp.zeros((16,), F32) for _ in range(NV))
        fin = lax.fori_loop(0, nchunks, chunk_body, init)
        d_last = fin[0]

        @pl.when(d_last >= 0)
        def _final_flush():
            for j in range(NV):
                accv[pl.ds((d_last - dlo) * F + 16 * j, 16)] = fin[1 + j]

        TR = PADN - 15 * RMAX

        @pl.when(nrows == RMAX)
        def _out_full():
            pltpu.sync_copy(accv, out_hbm.at[pl.ds(dlo * F, RMAX * F)])

        @pl.when(nrows != RMAX)
        def _out_tail():
            pltpu.sync_copy(accv.at[pl.ds(0, TR * F)],
                            out_hbm.at[pl.ds(dlo * F, TR * F)])

    return body


def _agg(y, src, dst, bnd):
    F = y.shape[1]
    mesh = plsc.VectorSubcoreMesh(core_axis_name="c", subcore_axis_name="s",
                                  num_cores=2, num_subcores=NSUB)
    f = pl.kernel(
        _mk_agg_body(F),
        out_type=jax.ShapeDtypeStruct((NP * F,), F32),
        mesh=mesh,
        compiler_params=pltpu.CompilerParams(use_tc_tiling_on_sc=False),
        scratch_types=[
            pltpu.VMEM((C,), jnp.int32),
            pltpu.VMEM((C + 16,), jnp.int32),
            pltpu.VMEM((48,), jnp.int32),
            pltpu.VMEM((C, F), F32),
            pltpu.VMEM((RMAX * F,), F32),
        ],
    )
    return f(y, src, dst, bnd).reshape(NP, F)


def kernel(x_p, x_d, edge_attr_p, edge_attr_d, edge_index_p, edge_index_d,
           x_p_batch, x_d_batch, params):
    gp, gd = params["gin_p"], params["gin_d"]
    w1 = [jnp.stack([gp[l]["W1"], gd[l]["W1"]]) for l in range(4)]
    b1 = [jnp.stack([gp[l]["b1"], gd[l]["b1"]])[:, None, :] for l in range(4)]
    w2 = [jnp.stack([gp[l]["W2"], gd[l]["W2"]]) for l in range(4)]
    b2 = [jnp.stack([gp[l]["b2"], gd[l]["b2"]])[:, None, :] for l in range(4)]

    x = jnp.concatenate([
        jnp.pad(x_p, ((0, PADN - NNET), (0, 0))),
        jnp.pad(x_d, ((0, PADN - NNET), (0, 0))),
    ], axis=0)

    dp, sp = lax.sort([edge_index_p[1], edge_index_p[0]], num_keys=1, is_stable=True)
    dd, sd = lax.sort([edge_index_d[1], edge_index_d[0]], num_keys=1, is_stable=True)
    fill_d = jnp.full((EPAD - EALL,), NP, jnp.int32)
    fill_s = jnp.zeros((EPAD - EALL,), jnp.int32)
    src = jnp.concatenate([sp, sd + PADN, fill_s])
    dst = jnp.concatenate([dp, dd + PADN, fill_d])


    dlos = (jnp.arange(32, dtype=jnp.int32) // NSUB) * PADN \
        + (jnp.arange(32, dtype=jnp.int32) % NSUB) * RMAX
    bnd = jnp.concatenate([
        jnp.searchsorted(dst[:EALL], dlos).astype(jnp.int32),
        jnp.array([EALL], jnp.int32),
        jnp.zeros((15,), jnp.int32),
    ])

    pad_ids = jnp.full((PADN - NNET,), NG, jnp.int32)
    bat = jnp.concatenate([x_p_batch, pad_ids, x_d_batch + 256, pad_ids])
    bat = bat.reshape(GRID, 1, RB)

    for l in range(4):
        agg = _agg(x, src, dst, bnd)
        x = _mid(x, agg, w1[l], b1[l], w2[l], b2[l], last=(l == 3))

    lins = params["lins"]
    fin = params["final"]
    return _pool_head(x, bat,
                      lins[0]["W"], lins[0]["b"][None, :],
                      lins[1]["W"], lins[1]["b"][None, :],
                      lins[2]["W"], lins[2]["b"][None, :],
                      fin["W"], fin["b"][None, :])
